# baseline (speedup 1.0000x reference)
import functools

import jax
import jax.numpy as jnp
from jax import lax
from jax.experimental import pallas as pl
from jax.experimental.pallas import tpu as pltpu
from jax.experimental.pallas import tpu_sc as plsc

NUM_EMBEDDINGS = 1000000
EMBEDDING_DIM = 64
BATCH = 16384 * 20
PAIR_W = 2 * EMBEDDING_DIM

_INFO = plsc.get_sparse_core_info()
NC = _INFO.num_cores
NS = _INFO.num_subcores
NW = NC * NS
BPW = BATCH // NW

IDX_W = 128
CHUNK = 256
NG = CHUNK // IDX_W
NCHUNKS = BPW // CHUNK
NPAIRS = NCHUNKS // 2
IDX_ROWS = BPW // IDX_W

_mesh = plsc.VectorSubcoreMesh(core_axis_name="c", subcore_axis_name="s")


@functools.partial(
    pl.kernel,
    mesh=_mesh,
    compiler_params=pltpu.CompilerParams(use_tc_tiling_on_sc=True),
    out_type=jax.ShapeDtypeStruct((BATCH, PAIR_W), jnp.float32),
    scratch_types=[
        pltpu.VMEM((IDX_ROWS, IDX_W), jnp.int32),
        pltpu.VMEM((2, CHUNK, PAIR_W), jnp.float32),
        pltpu.SemaphoreType.DMA,
        pltpu.SemaphoreType.DMA,
    ],
)
def _gather_kernel(pidx_hbm, wpair_hbm, out_hbm, idx_v, rows_v, gsem, osem):
    wid = lax.axis_index("s") * NC + lax.axis_index("c")
    base = wid * BPW
    idx_row_base = pl.multiple_of(wid * IDX_ROWS, 8)

    pltpu.sync_copy(pidx_hbm.at[pl.ds(idx_row_base, IDX_ROWS)], idx_v)

    def out_slice(g):
        o = pl.multiple_of(base + g * CHUNK, CHUNK)
        return out_hbm.at[pl.ds(o, CHUNK)]

    def fire_gathers(g, b):
        for j in range(NG):
            pltpu.async_copy(
                wpair_hbm.at[idx_v.at[g * NG + j]],
                rows_v.at[b].at[pl.ds(j * IDX_W, IDX_W)],
                gsem,
            )

    def wait_gathers(g, b):
        for j in range(NG):
            pltpu.make_async_copy(
                wpair_hbm.at[idx_v.at[g * NG + j]],
                rows_v.at[b].at[pl.ds(j * IDX_W, IDX_W)],
                gsem,
            ).wait()

    def start_out(g, b):
        pltpu.async_copy(rows_v.at[b], out_slice(g), osem)

    def wait_out(g, b):
        pltpu.make_async_copy(rows_v.at[b], out_slice(g), osem).wait()

    def chunk(g, b, do_wait_out):
        if do_wait_out:
            wait_out(g - 2, b)
        fire_gathers(g, b)
        wait_gathers(g, b)
        start_out(g, b)

    chunk(0, 0, False)
    chunk(1, 1, False)

    def body(t, carry):
        g0 = 2 * t
        chunk(g0, 0, True)
        chunk(g0 + 1, 1, True)
        return carry

    lax.fori_loop(1, NPAIRS, body, 0)

    wait_out(NCHUNKS - 2, 0)
    wait_out(NCHUNKS - 1, 1)


def kernel(input_, weight):
    idx = input_.reshape(BATCH)
    pidx = (idx >> 1).reshape(BATCH // IDX_W, IDX_W)
    wpair = weight.reshape(NUM_EMBEDDINGS // 2, PAIR_W)
    pairs = _gather_kernel(pidx, wpair)
    half = pairs.reshape(BATCH, 2, EMBEDDING_DIM)
    out = jnp.where((idx & 1)[:, None] == 0, half[:, 0, :], half[:, 1, :])
    return out.reshape(input_.shape[0], input_.shape[1], EMBEDDING_DIM)


# speedup vs baseline: 1.5028x; 1.5028x over previous
import functools

import jax
import jax.numpy as jnp
from jax import lax
from jax.experimental import pallas as pl
from jax.experimental.pallas import tpu as pltpu
from jax.experimental.pallas import tpu_sc as plsc

NUM_EMBEDDINGS = 1000000
EMBEDDING_DIM = 64
BATCH = 16384 * 20
PAIR_W = 2 * EMBEDDING_DIM

_INFO = plsc.get_sparse_core_info()
NC = _INFO.num_cores
NS = _INFO.num_subcores
NW = NC * NS
BPW = BATCH // NW

IDX_W = 128
CHUNK = 256
NG = CHUNK // IDX_W
NCHUNKS = BPW // CHUNK
NPAIRS = NCHUNKS // 2
IDX_ROWS = BPW // IDX_W

_mesh = plsc.VectorSubcoreMesh(core_axis_name="c", subcore_axis_name="s")


@functools.partial(
    pl.kernel,
    mesh=_mesh,
    compiler_params=pltpu.CompilerParams(use_tc_tiling_on_sc=True),
    out_type=jax.ShapeDtypeStruct((BATCH, PAIR_W), jnp.float32),
    scratch_types=[
        pltpu.VMEM((IDX_ROWS, IDX_W), jnp.int32),
        pltpu.VMEM((2, CHUNK, PAIR_W), jnp.float32),
        pltpu.SemaphoreType.DMA,
        pltpu.SemaphoreType.DMA,
    ],
)
def _gather_kernel(pidx_hbm, wpair_hbm, out_hbm, idx_v, rows_v, gsem, osem):
    wid = lax.axis_index("s") * NC + lax.axis_index("c")
    base = wid * BPW
    idx_row_base = pl.multiple_of(wid * IDX_ROWS, 8)

    pltpu.sync_copy(pidx_hbm.at[pl.ds(idx_row_base, IDX_ROWS)], idx_v)

    def out_slice(g):
        o = pl.multiple_of(base + g * CHUNK, CHUNK)
        return out_hbm.at[pl.ds(o, CHUNK)]

    def fire_gathers(g, b):
        for j in range(NG):
            pltpu.async_copy(
                wpair_hbm.at[idx_v.at[g * NG + j]],
                rows_v.at[b].at[pl.ds(j * IDX_W, IDX_W)],
                gsem,
            )

    def wait_gathers(g, b):
        for j in range(NG):
            pltpu.make_async_copy(
                wpair_hbm.at[idx_v.at[g * NG + j]],
                rows_v.at[b].at[pl.ds(j * IDX_W, IDX_W)],
                gsem,
            ).wait()

    def start_out(g, b):
        pltpu.async_copy(rows_v.at[b], out_slice(g), osem)

    def wait_out(g, b):
        pltpu.make_async_copy(rows_v.at[b], out_slice(g), osem).wait()

    def chunk(g, b, do_wait_out):
        if do_wait_out:
            wait_out(g - 2, b)
        fire_gathers(g, b)
        wait_gathers(g, b)
        start_out(g, b)

    chunk(0, 0, False)
    chunk(1, 1, False)

    def body(t, carry):
        g0 = 2 * t
        chunk(g0, 0, True)
        chunk(g0 + 1, 1, True)
        return carry

    lax.fori_loop(1, NPAIRS, body, 0)

    wait_out(NCHUNKS - 2, 0)
    wait_out(NCHUNKS - 1, 1)


def kernel(input_, weight):
    idx = input_.reshape(BATCH)
    pidx = (idx >> 1).reshape(BATCH // IDX_W, IDX_W)
    wpair = weight.reshape(NUM_EMBEDDINGS // 2, PAIR_W)
    pairs = _gather_kernel(pidx, wpair)
    lo = lax.slice(pairs, (0, 0), (BATCH, EMBEDDING_DIM))
    hi = lax.slice(pairs, (0, EMBEDDING_DIM), (BATCH, PAIR_W))
    out = jnp.where((idx & 1)[:, None] == 0, lo, hi)
    return out.reshape(input_.shape[0], input_.shape[1], EMBEDDING_DIM)
